# Initial kernel scaffold; baseline (speedup 1.0000x reference)
import jax
import jax.numpy as jnp
from jax.experimental import pallas as pl


def kernel(z, edge_index, edge_type, rel_emb):
    raise NotImplementedError("write your pallas kernel here")


# speedup vs baseline: 3.8394x; 3.8394x over previous
import functools

import jax
import jax.numpy as jnp
from jax import lax
from jax.experimental import pallas as pl
from jax.experimental.pallas import tpu as pltpu
from jax.experimental.pallas import tpu_sc as plsc

N_NODES = 10000
N_EDGES = 320000
D = 128
NUM_REL = 1000

NC = 2
NS = 16
NW = NC * NS
EPW = N_EDGES // NW
C = 80
NCHUNK = EPW // C

RB = 2000


def _sc_body(z, head, tail, etype, rel_emb, out,
             hidx, tidx, ridx, hrows, trows, rrows, obuf, sem):
    wid = lax.axis_index("s") * NC + lax.axis_index("c")
    base = wid * EPW

    pltpu.sync_copy(head.at[pl.ds(base, EPW)], hidx)
    pltpu.sync_copy(tail.at[pl.ds(base, EPW)], tidx)
    pltpu.sync_copy(etype.at[pl.ds(base, EPW)], ridx)

    def chunk(g, carry):
        off = g * C
        ch = pltpu.async_copy(z.at[hidx.at[pl.ds(off, C)]], hrows, sem)
        ct = pltpu.async_copy(z.at[tidx.at[pl.ds(off, C)]], trows, sem)
        cr = pltpu.async_copy(rel_emb.at[ridx.at[pl.ds(off, C)]], rrows, sem)
        ch.wait()
        ct.wait()
        cr.wait()

        def edge(e, c2):
            acc = hrows[e, pl.ds(0, 16)] * rrows[e, pl.ds(0, 16)] * trows[e, pl.ds(0, 16)]
            for j in range(1, D // 16):
                acc = acc + (hrows[e, pl.ds(j * 16, 16)]
                             * rrows[e, pl.ds(j * 16, 16)]
                             * trows[e, pl.ds(j * 16, 16)])
            obuf[e, :] = acc
            return c2

        lax.fori_loop(0, C, edge, 0)
        pltpu.sync_copy(obuf, out.at[pl.ds(base + off, C), :])
        return carry

    lax.fori_loop(0, NCHUNK, chunk, 0)


def _tc_body(x_ref, o_ref):
    seg = jnp.equal(
        lax.broadcasted_iota(jnp.int32, (128, 8), 0) // 16,
        lax.broadcasted_iota(jnp.int32, (128, 8), 1),
    ).astype(jnp.float32)
    o_ref[...] = jax.lax.dot_general(
        x_ref[...], seg, (((1,), (0,)), ((), ())),
        preferred_element_type=jnp.float32)


@jax.jit
def kernel(z, edge_index, edge_type, rel_emb):
    head = edge_index[0]
    tail = edge_index[1]
    mesh = plsc.VectorSubcoreMesh(core_axis_name="c", subcore_axis_name="s")
    sc_k = functools.partial(
        pl.kernel,
        out_type=jax.ShapeDtypeStruct((N_EDGES, 16), jnp.float32),
        mesh=mesh,
        scratch_types=[
            pltpu.VMEM((EPW,), jnp.int32),
            pltpu.VMEM((EPW,), jnp.int32),
            pltpu.VMEM((EPW,), jnp.int32),
            pltpu.VMEM((C, D), jnp.float32),
            pltpu.VMEM((C, D), jnp.float32),
            pltpu.VMEM((C, D), jnp.float32),
            pltpu.VMEM((C, 16), jnp.float32),
            pltpu.SemaphoreType.DMA,
        ],
    )(_sc_body)
    partial_sums = sc_k(z, head, tail, edge_type, rel_emb)

    xr = partial_sums.reshape(N_EDGES // 8, 128)
    out2 = pl.pallas_call(
        _tc_body,
        out_shape=jax.ShapeDtypeStruct((N_EDGES // 8, 8), jnp.float32),
        grid=(N_EDGES // 8 // RB,),
        in_specs=[pl.BlockSpec((RB, 128), lambda i: (i, 0))],
        out_specs=pl.BlockSpec((RB, 8), lambda i: (i, 0)),
    )(xr)
    return out2.reshape(N_EDGES)


# speedup vs baseline: 5.2450x; 1.3661x over previous
import functools

import jax
import jax.numpy as jnp
from jax import lax
from jax.experimental import pallas as pl
from jax.experimental.pallas import tpu as pltpu
from jax.experimental.pallas import tpu_sc as plsc

N_NODES = 10000
N_EDGES = 320000
D = 128
NUM_REL = 1000

NC = 2
NS = 16
NW = NC * NS
EPW = N_EDGES // NW
C = 80
NCHUNK = EPW // C

RB = 2000


def _sc_body(z, head, tail, etype, rel_emb, out,
             hidx, tidx, ridx,
             hrows0, trows0, rrows0, obuf0,
             hrows1, trows1, rrows1, obuf1,
             gsem0, gsem1, osem0, osem1):
    wid = lax.axis_index("s") * NC + lax.axis_index("c")
    base = wid * EPW

    bufs = ((hrows0, trows0, rrows0, obuf0, gsem0, osem0),
            (hrows1, trows1, rrows1, obuf1, gsem1, osem1))

    pltpu.sync_copy(head.at[pl.ds(base, EPW)], hidx)
    pltpu.sync_copy(tail.at[pl.ds(base, EPW)], tidx)
    pltpu.sync_copy(etype.at[pl.ds(base, EPW)], ridx)

    def gather_descs(g, p):
        off = g * C
        hb, tb, rb, _, gs, _ = bufs[p]
        return (
            pltpu.make_async_copy(z.at[hidx.at[pl.ds(off, C)]], hb, gs),
            pltpu.make_async_copy(z.at[tidx.at[pl.ds(off, C)]], tb, gs),
            pltpu.make_async_copy(rel_emb.at[ridx.at[pl.ds(off, C)]], rb, gs),
        )

    def fire_gathers(g, p):
        for d in gather_descs(g, p):
            d.start()

    def wait_gathers(g, p):
        for d in gather_descs(g, p):
            d.wait()

    def out_desc(g, p):
        _, _, _, ob, _, osm = bufs[p]
        return pltpu.make_async_copy(ob, out.at[pl.ds(base + g * C, C), :], osm)

    def compute(g, p):
        hb, tb, rb, ob, _, _ = bufs[p]

        def group(g8, c2):
            e0 = g8 * 8
            for k in range(8):
                e = e0 + k
                acc = hb[e, pl.ds(0, 16)] * rb[e, pl.ds(0, 16)] * tb[e, pl.ds(0, 16)]
                for j in range(1, D // 16):
                    acc = acc + (hb[e, pl.ds(j * 16, 16)]
                                 * rb[e, pl.ds(j * 16, 16)]
                                 * tb[e, pl.ds(j * 16, 16)])
                ob[e, :] = acc
            return c2

        lax.fori_loop(0, C // 8, group, 0)

    def step(g, p, fire_next):
        wait_gathers(g, p)
        if fire_next:
            fire_gathers(g + 1, p ^ 1)
        out_desc(g, p).wait()
        compute(g, p)
        out_desc(g, p).start()

    fire_gathers(0, 0)
    out_desc(0, 0).start()
    out_desc(1, 1).start()

    def pair(i, carry):
        step(2 * i, 0, True)
        step(2 * i + 1, 1, True)
        return carry

    lax.fori_loop(0, (NCHUNK - 1) // 2, pair, 0)
    step(NCHUNK - 1, 0, False)
    out_desc(NCHUNK - 1, 0).wait()
    out_desc(NCHUNK - 2, 1).wait()


def _tc_body(x_ref, o_ref):
    seg = jnp.equal(
        lax.broadcasted_iota(jnp.int32, (128, 8), 0) // 16,
        lax.broadcasted_iota(jnp.int32, (128, 8), 1),
    ).astype(jnp.float32)
    o_ref[...] = jax.lax.dot_general(
        x_ref[...], seg, (((1,), (0,)), ((), ())),
        preferred_element_type=jnp.float32)


@jax.jit
def kernel(z, edge_index, edge_type, rel_emb):
    head = edge_index[0]
    tail = edge_index[1]
    mesh = plsc.VectorSubcoreMesh(core_axis_name="c", subcore_axis_name="s")
    sc_k = functools.partial(
        pl.kernel,
        out_type=jax.ShapeDtypeStruct((N_EDGES, 16), jnp.float32),
        mesh=mesh,
        scratch_types=[
            pltpu.VMEM((EPW,), jnp.int32),
            pltpu.VMEM((EPW,), jnp.int32),
            pltpu.VMEM((EPW,), jnp.int32),
            pltpu.VMEM((C, D), jnp.float32),
            pltpu.VMEM((C, D), jnp.float32),
            pltpu.VMEM((C, D), jnp.float32),
            pltpu.VMEM((C, 16), jnp.float32),
            pltpu.VMEM((C, D), jnp.float32),
            pltpu.VMEM((C, D), jnp.float32),
            pltpu.VMEM((C, D), jnp.float32),
            pltpu.VMEM((C, 16), jnp.float32),
            pltpu.SemaphoreType.DMA,
            pltpu.SemaphoreType.DMA,
            pltpu.SemaphoreType.DMA,
            pltpu.SemaphoreType.DMA,
        ],
    )(_sc_body)
    partial_sums = sc_k(z, head, tail, edge_type, rel_emb)

    xr = partial_sums.reshape(N_EDGES // 8, 128)
    out2 = pl.pallas_call(
        _tc_body,
        out_shape=jax.ShapeDtypeStruct((N_EDGES // 8, 8), jnp.float32),
        grid=(N_EDGES // 8 // RB,),
        in_specs=[pl.BlockSpec((RB, 128), lambda i: (i, 0))],
        out_specs=pl.BlockSpec((RB, 8), lambda i: (i, 0)),
    )(xr)
    return out2.reshape(N_EDGES)


# speedup vs baseline: 7.2030x; 1.3733x over previous
import functools

import jax
import jax.numpy as jnp
from jax import lax
from jax.experimental import pallas as pl
from jax.experimental.pallas import tpu as pltpu
from jax.experimental.pallas import tpu_sc as plsc

N_NODES = 10000
N_EDGES = 320000
D = 128
NUM_REL = 1000

NC = 2
NS = 16
NW = NC * NS
EPW = N_EDGES // NW
C = 80
NCHUNK = EPW // C

RB = 2000


def _sc_body(z, head, tail, etype, rel_emb, out,
             hidx, tidx, ridx,
             hrows0, trows0, rrows0, obuf0,
             hrows1, trows1, rrows1, obuf1,
             gsem0, gsem1, osem0, osem1):
    wid = lax.axis_index("s") * NC + lax.axis_index("c")
    base = wid * EPW

    bufs = ((hrows0, trows0, rrows0, obuf0, gsem0, osem0),
            (hrows1, trows1, rrows1, obuf1, gsem1, osem1))

    pltpu.sync_copy(head.at[pl.ds(base, EPW)], hidx)
    pltpu.sync_copy(tail.at[pl.ds(base, EPW)], tidx)
    pltpu.sync_copy(etype.at[pl.ds(base, EPW)], ridx)

    def gather_descs(g, p):
        off = g * C
        hb, tb, rb, _, gs, _ = bufs[p]
        return (
            pltpu.make_async_copy(z.at[hidx.at[pl.ds(off, C)]], hb, gs),
            pltpu.make_async_copy(z.at[tidx.at[pl.ds(off, C)]], tb, gs),
            pltpu.make_async_copy(rel_emb.at[ridx.at[pl.ds(off, C)]], rb, gs),
        )

    def fire_gathers(g, p):
        for d in gather_descs(g, p):
            d.start()

    def wait_gathers(g, p):
        for d in gather_descs(g, p):
            d.wait()

    def out_desc(g, p):
        _, _, _, ob, _, osm = bufs[p]
        return pltpu.make_async_copy(ob, out.at[pl.ds(base + g * C, C), :], osm)

    def compute(g, p):
        hb, tb, rb, ob, _, _ = bufs[p]

        def group(g8, c2):
            e0 = g8 * 8
            for k in range(8):
                e = e0 + k
                acc = hb[e, pl.ds(0, 16)] * rb[e, pl.ds(0, 16)] * tb[e, pl.ds(0, 16)]
                for j in range(1, D // 16):
                    acc = acc + (hb[e, pl.ds(j * 16, 16)]
                                 * rb[e, pl.ds(j * 16, 16)]
                                 * tb[e, pl.ds(j * 16, 16)])
                ob[e, :] = acc
            return c2

        lax.fori_loop(0, C // 8, group, 0)

    def step(g, p, fire_next):
        wait_gathers(g, p)
        if fire_next:
            fire_gathers(g + 1, p ^ 1)
        out_desc(g, p).wait()
        compute(g, p)
        out_desc(g, p).start()

    fire_gathers(0, 0)
    out_desc(0, 0).start()
    out_desc(1, 1).start()

    def pair(i, carry):
        step(2 * i, 0, True)
        step(2 * i + 1, 1, True)
        return carry

    lax.fori_loop(0, (NCHUNK - 1) // 2, pair, 0)
    step(NCHUNK - 1, 0, False)
    out_desc(NCHUNK - 1, 0).wait()
    out_desc(NCHUNK - 2, 1).wait()


def _tc_body(x_ref, o_ref):
    seg = jnp.equal(
        lax.broadcasted_iota(jnp.int32, (128, 8), 0) // 16,
        lax.broadcasted_iota(jnp.int32, (128, 8), 1),
    ).astype(jnp.float32)
    o_ref[...] = jax.lax.dot_general(
        x_ref[...], seg, (((1,), (0,)), ((), ())),
        preferred_element_type=jnp.float32)


@jax.jit
def kernel(z, edge_index, edge_type, rel_emb):
    head = edge_index[0]
    tail = edge_index[1]
    mesh = plsc.VectorSubcoreMesh(core_axis_name="c", subcore_axis_name="s")
    sc_k = functools.partial(
        pl.kernel,
        out_type=jax.ShapeDtypeStruct((N_EDGES, 16), jnp.float32),
        mesh=mesh,
        compiler_params=pltpu.CompilerParams(use_tc_tiling_on_sc=False),
        scratch_types=[
            pltpu.VMEM((EPW,), jnp.int32),
            pltpu.VMEM((EPW,), jnp.int32),
            pltpu.VMEM((EPW,), jnp.int32),
            pltpu.VMEM((C, D), jnp.float32),
            pltpu.VMEM((C, D), jnp.float32),
            pltpu.VMEM((C, D), jnp.float32),
            pltpu.VMEM((C, 16), jnp.float32),
            pltpu.VMEM((C, D), jnp.float32),
            pltpu.VMEM((C, D), jnp.float32),
            pltpu.VMEM((C, D), jnp.float32),
            pltpu.VMEM((C, 16), jnp.float32),
            pltpu.SemaphoreType.DMA,
            pltpu.SemaphoreType.DMA,
            pltpu.SemaphoreType.DMA,
            pltpu.SemaphoreType.DMA,
        ],
    )(_sc_body)
    partial_sums = sc_k(z, head, tail, edge_type, rel_emb)

    xr = partial_sums.reshape(N_EDGES // 8, 128)
    out2 = pl.pallas_call(
        _tc_body,
        out_shape=jax.ShapeDtypeStruct((N_EDGES // 8, 8), jnp.float32),
        grid=(N_EDGES // 8 // RB,),
        in_specs=[pl.BlockSpec((RB, 128), lambda i: (i, 0))],
        out_specs=pl.BlockSpec((RB, 8), lambda i: (i, 0)),
    )(xr)
    return out2.reshape(N_EDGES)


# speedup vs baseline: 7.6153x; 1.0572x over previous
import functools

import jax
import jax.numpy as jnp
from jax import lax
from jax.experimental import pallas as pl
from jax.experimental.pallas import tpu as pltpu
from jax.experimental.pallas import tpu_sc as plsc

N_NODES = 10000
N_EDGES = 320000
D = 128
NUM_REL = 1000

NC = 2
NS = 16
NW = NC * NS
EPW = N_EDGES // NW
C = 80
NCHUNK = EPW // C

RB = 8000


def _sc_body(z, eidx, etype, rel_emb, out,
             hidx, tidx, ridx,
             hrows0, trows0, rrows0, obuf0,
             hrows1, trows1, rrows1, obuf1,
             gsem0, gsem1, osem0, osem1):
    wid = lax.axis_index("s") * NC + lax.axis_index("c")
    base = wid * EPW

    bufs = ((hrows0, trows0, rrows0, obuf0, gsem0, osem0),
            (hrows1, trows1, rrows1, obuf1, gsem1, osem1))

    pltpu.sync_copy(eidx.at[0, pl.ds(base, EPW)], hidx)
    pltpu.sync_copy(eidx.at[1, pl.ds(base, EPW)], tidx)
    pltpu.sync_copy(etype.at[pl.ds(base, EPW)], ridx)

    def gather_descs(g, p):
        off = g * C
        hb, tb, rb, _, gs, _ = bufs[p]
        return (
            pltpu.make_async_copy(z.at[hidx.at[pl.ds(off, C)]], hb, gs),
            pltpu.make_async_copy(z.at[tidx.at[pl.ds(off, C)]], tb, gs),
            pltpu.make_async_copy(rel_emb.at[ridx.at[pl.ds(off, C)]], rb, gs),
        )

    def fire_gathers(g, p):
        for d in gather_descs(g, p):
            d.start()

    def wait_gathers(g, p):
        for d in gather_descs(g, p):
            d.wait()

    def out_desc(g, p):
        _, _, _, ob, _, osm = bufs[p]
        return pltpu.make_async_copy(ob, out.at[pl.ds(base + g * C, C), :], osm)

    def compute(g, p):
        hb, tb, rb, ob, _, _ = bufs[p]

        def group(g8, c2):
            e0 = g8 * 8
            for k in range(8):
                e = e0 + k
                acc = hb[e, pl.ds(0, 16)] * rb[e, pl.ds(0, 16)] * tb[e, pl.ds(0, 16)]
                for j in range(1, D // 16):
                    acc = acc + (hb[e, pl.ds(j * 16, 16)]
                                 * rb[e, pl.ds(j * 16, 16)]
                                 * tb[e, pl.ds(j * 16, 16)])
                ob[e, :] = acc
            return c2

        lax.fori_loop(0, C // 8, group, 0)

    def step(g, p, fire_next):
        wait_gathers(g, p)
        if fire_next:
            fire_gathers(g + 1, p ^ 1)
        out_desc(g, p).wait()
        compute(g, p)
        out_desc(g, p).start()

    fire_gathers(0, 0)
    out_desc(0, 0).start()
    out_desc(1, 1).start()

    def pair(i, carry):
        step(2 * i, 0, True)
        step(2 * i + 1, 1, True)
        return carry

    lax.fori_loop(0, (NCHUNK - 1) // 2, pair, 0)
    step(NCHUNK - 1, 0, False)
    out_desc(NCHUNK - 1, 0).wait()
    out_desc(NCHUNK - 2, 1).wait()


def _tc_body(x_ref, o_ref):
    seg = jnp.equal(
        lax.broadcasted_iota(jnp.int32, (128, 8), 0) // 16,
        lax.broadcasted_iota(jnp.int32, (128, 8), 1),
    ).astype(jnp.float32)
    o_ref[...] = jax.lax.dot_general(
        x_ref[...], seg, (((1,), (0,)), ((), ())),
        preferred_element_type=jnp.float32)


@jax.jit
def kernel(z, edge_index, edge_type, rel_emb):
    mesh = plsc.VectorSubcoreMesh(core_axis_name="c", subcore_axis_name="s")
    sc_k = functools.partial(
        pl.kernel,
        out_type=jax.ShapeDtypeStruct((N_EDGES, 16), jnp.float32),
        mesh=mesh,
        compiler_params=pltpu.CompilerParams(use_tc_tiling_on_sc=False),
        scratch_types=[
            pltpu.VMEM((EPW,), jnp.int32),
            pltpu.VMEM((EPW,), jnp.int32),
            pltpu.VMEM((EPW,), jnp.int32),
            pltpu.VMEM((C, D), jnp.float32),
            pltpu.VMEM((C, D), jnp.float32),
            pltpu.VMEM((C, D), jnp.float32),
            pltpu.VMEM((C, 16), jnp.float32),
            pltpu.VMEM((C, D), jnp.float32),
            pltpu.VMEM((C, D), jnp.float32),
            pltpu.VMEM((C, D), jnp.float32),
            pltpu.VMEM((C, 16), jnp.float32),
            pltpu.SemaphoreType.DMA,
            pltpu.SemaphoreType.DMA,
            pltpu.SemaphoreType.DMA,
            pltpu.SemaphoreType.DMA,
        ],
    )(_sc_body)
    partial_sums = sc_k(z, edge_index, edge_type, rel_emb)

    xr = partial_sums.reshape(N_EDGES // 8, 128)
    out2 = pl.pallas_call(
        _tc_body,
        out_shape=jax.ShapeDtypeStruct((N_EDGES // 8, 8), jnp.float32),
        grid=(N_EDGES // 8 // RB,),
        in_specs=[pl.BlockSpec((RB, 128), lambda i: (i, 0))],
        out_specs=pl.BlockSpec((RB, 8), lambda i: (i, 0)),
    )(xr)
    return out2.reshape(N_EDGES)
